# baseline (speedup 1.0000x reference)
import functools

import jax
import jax.numpy as jnp
from jax import lax
from jax.experimental import pallas as pl
from jax.experimental.pallas import tpu as pltpu
from jax.experimental.pallas import tpu_sc as plsc

NC, NS = 2, 16
NW = NC * NS
B, T, D, NREL = 4096, 32, 64, 200
NSETS = 4
N = NSETS * T * B
RPW = N // NW
GCH = 4
GIT = RPW // (GCH * 128)

BB = 2048
NB = B // BB

CB = 4
NCH = (B // NW) // CB

_SC_PARAMS = dict(
    mesh=plsc.VectorSubcoreMesh(core_axis_name="c", subcore_axis_name="s"),
    compiler_params=pltpu.CompilerParams(
        use_tc_tiling_on_sc=False, needs_layout_passes=False),
)


def _wid():
    return lax.axis_index("s") * NC + lax.axis_index("c")


NENT = 1000000
EB = 8192
NPKB = (NENT + EB - 1) // EB


def _repack_body(et_ref, out_ref, outb_ref):
    ta = jnp.transpose(et_ref[:, 0:EB // 2])
    tb = jnp.transpose(et_ref[:, EB // 2:EB])
    out_ref[:, 0:D] = ta
    out_ref[:, D:2 * D] = tb
    outb_ref[:, 0:D] = ta.astype(jnp.bfloat16)
    outb_ref[:, D:2 * D] = tb.astype(jnp.bfloat16)


def _repack_call(et):
    return pl.pallas_call(
        _repack_body,
        grid=(NPKB,),
        in_specs=[pl.BlockSpec((D, EB), lambda i: (0, i))],
        out_specs=[pl.BlockSpec((EB // 2, 2 * D), lambda i: (i, 0)),
                   pl.BlockSpec((EB // 2, 2 * D), lambda i: (i, 0))],
        out_shape=[
            jax.ShapeDtypeStruct((NPKB * (EB // 2), 2 * D), jnp.float32),
            jax.ShapeDtypeStruct((NPKB * (EB // 2), 2 * D), jnp.bfloat16),
        ],
    )(et)


def _pidx(e):
    g = e // EB
    l = e % EB
    return 2 * (g * (EB // 2) + l % (EB // 2)) + l // (EB // 2)


def _r1_body(rel_ref, w1_ref, out_ref):
    out_ref[...] = jnp.dot(rel_ref[...], w1_ref[D:2 * D, :],
                           preferred_element_type=jnp.float32,
                           precision=lax.Precision.HIGHEST).astype(jnp.bfloat16)


def _r1_call(rel, w1):
    return pl.pallas_call(
        _r1_body,
        out_shape=jax.ShapeDtypeStruct((NREL, D), jnp.bfloat16),
    )(rel, w1)


CHN = GCH * 128


@functools.partial(
    pl.kernel,
    out_type=jax.ShapeDtypeStruct((N, D), jnp.bfloat16),
    scratch_types=[
        pltpu.VMEM((CHN,), jnp.int32),
        pltpu.VMEM((CHN,), jnp.int32),
        pltpu.VMEM((CHN, D), jnp.bfloat16),
        pltpu.VMEM((CHN, D), jnp.bfloat16),
        pltpu.SemaphoreType.DMA,
        pltpu.SemaphoreType.DMA,
        pltpu.SemaphoreType.DMA,
    ],
    **_SC_PARAMS,
)
def _gather_rows(tbl, idx, out, ix0, ix1, rv0, rv1, semg, semw0, semw1):
    w = _wid()
    base = w * RPW
    last = base + (GIT - 1) * CHN

    def fire(ixb, rvb):
        return [pltpu.async_copy(tbl.at[ixb.at[pl.ds(128 * k, 128)]],
                                 rvb.at[pl.ds(128 * k, 128)], semg)
                for k in range(GCH)]

    pltpu.sync_copy(idx.at[pl.ds(base, CHN)], ix0)
    c0 = fire(ix0, rv0)
    pltpu.sync_copy(idx.at[pl.ds(base + CHN, CHN)], ix1)
    for c in c0:
        c.wait()
    pltpu.async_copy(rv0, out.at[pl.ds(base, CHN)], semw0)
    c1 = fire(ix1, rv1)
    pltpu.sync_copy(idx.at[pl.ds(base + 2 * CHN, CHN)], ix0)
    for c in c1:
        c.wait()
    pltpu.async_copy(rv1, out.at[pl.ds(base + CHN, CHN)], semw1)

    def step(m, carry):
        g0 = base + 2 * m * CHN
        g1 = g0 + CHN
        pltpu.make_async_copy(rv0, out.at[pl.ds(g0, CHN)], semw0).wait()
        ca = fire(ix0, rv0)
        pltpu.sync_copy(idx.at[pl.ds(g1, CHN)], ix1)
        for c in ca:
            c.wait()
        pltpu.async_copy(rv0, out.at[pl.ds(g0, CHN)], semw0)
        pltpu.make_async_copy(rv1, out.at[pl.ds(g1, CHN)], semw1).wait()
        cb = fire(ix1, rv1)
        nxt2 = jnp.minimum(g1 + CHN, last)
        pltpu.sync_copy(idx.at[pl.ds(nxt2, CHN)], ix0)
        for c in cb:
            c.wait()
        pltpu.async_copy(rv1, out.at[pl.ds(g1, CHN)], semw1)
        return carry

    lax.fori_loop(1, GIT // 2, step, 0)
    pltpu.make_async_copy(rv0, out.at[pl.ds(last - CHN, CHN)], semw0).wait()
    pltpu.make_async_copy(rv1, out.at[pl.ds(last, CHN)], semw1).wait()


def _att_body(gh_ref, gr_ref, w1d_ref, w2d_ref, w3d_ref,
              z_ref, uhm_ref, ihm_ref):
    j = pl.program_id(1)

    x = jnp.dot(gh_ref[...], w1d_ref[...],
                preferred_element_type=jnp.float32)
    x = jnp.maximum(x + gr_ref[...].astype(jnp.float32), 0.0)
    h = gh_ref[...].astype(jnp.float32)
    x = jnp.maximum(jnp.dot(x.astype(jnp.bfloat16), w2d_ref[...],
                            preferred_element_type=jnp.float32), 0.0)
    zp = jnp.dot(x.astype(jnp.bfloat16), w3d_ref[...],
                 preferred_element_type=jnp.float32)
    iot = lax.broadcasted_iota(jnp.int32, (1, 4 * D), 1)
    acc = (zp[:, 0:1] * (iot == j).astype(jnp.float32)
           + zp[:, 1:2] * (iot == j + 2 * D).astype(jnp.float32))

    @pl.when(j == 0)
    def _():
        z_ref[...] = acc

    @pl.when(j > 0)
    def _():
        z_ref[...] = z_ref[...] + acc

    inv = 1.0 / T

    @pl.when(j == 0)
    def _():
        uhm_ref[...] = h * inv

    @pl.when((j >= 1) & (j < T))
    def _():
        uhm_ref[...] = uhm_ref[...] + h * inv

    @pl.when(j == 2 * T)
    def _():
        ihm_ref[...] = h * inv

    @pl.when((j > 2 * T) & (j < 3 * T))
    def _():
        ihm_ref[...] = ihm_ref[...] + h * inv


def _att_call(ghp, grp, w1d, w2d, w3d):
    bbp = BB // 2
    return pl.pallas_call(
        _att_body,
        grid=(NB, NSETS * T),
        in_specs=[
            pl.BlockSpec((bbp, 2 * D), lambda i, j: (j * NB + i, 0)),
            pl.BlockSpec((bbp, 2 * D), lambda i, j: (j * NB + i, 0)),
            pl.BlockSpec((2 * D, 2 * D), lambda i, j: (0, 0)),
            pl.BlockSpec((2 * D, 2 * D), lambda i, j: (0, 0)),
            pl.BlockSpec((2 * D, 2), lambda i, j: (0, 0)),
        ],
        out_specs=[
            pl.BlockSpec((bbp, 4 * D), lambda i, j: (i, 0)),
            pl.BlockSpec((bbp, 2 * D), lambda i, j: (i, 0)),
            pl.BlockSpec((bbp, 2 * D), lambda i, j: (i, 0)),
        ],
        out_shape=[
            jax.ShapeDtypeStruct((B // 2, 4 * D), jnp.float32),
            jax.ShapeDtypeStruct((B // 2, 2 * D), jnp.float32),
            jax.ShapeDtypeStruct((B // 2, 2 * D), jnp.float32),
        ],
    )(ghp, grp, w1d, w2d, w3d)


@functools.partial(
    pl.kernel,
    out_type=jax.ShapeDtypeStruct((B,), jnp.float32),
    scratch_types=[
        pltpu.VMEM((NSETS * 128,), jnp.int32),
        pltpu.VMEM((2 * (B // NW),), jnp.int32),
        pltpu.VMEM((NSETS * CB * T,), jnp.float32),
        pltpu.VMEM((NSETS * CB * T, D), jnp.float32),
        pltpu.VMEM((CB, D), jnp.float32),
        pltpu.VMEM((CB * D,), jnp.float32),
        pltpu.VMEM((CB * D,), jnp.float32),
        pltpu.VMEM((16,), jnp.float32),
        pltpu.SemaphoreType.DMA,
    ],
    **_SC_PARAMS,
)
def _finisher(emb, tix, zb, uhm, ihm, items2, out,
              idx_v, iidx_v, zv, rows_v, gv, uv, iv, svec_ref, sem):
    w = _wid()
    bpw = B // NW
    pltpu.sync_copy(items2.at[pl.ds(w * 2 * bpw, 2 * bpw)], iidx_v)

    def chunk(ch, svec):
        b0 = w * bpw + ch * CB
        for s in range(NSETS):
            pltpu.sync_copy(tix.at[pl.ds(s * B * T + b0 * T, CB * T)],
                            idx_v.at[pl.ds(s * 128, 128)])
        pltpu.sync_copy(zb.at[pl.ds(b0 * 128, CB * 128)], zv)
        pltpu.sync_copy(uhm.at[pl.ds(b0 * D, CB * D)], uv)
        pltpu.sync_copy(ihm.at[pl.ds(b0 * D, CB * D)], iv)
        cps = [
            pltpu.async_copy(emb.at[idx_v.at[pl.ds(128 * k, 128)]],
                             rows_v.at[pl.ds(128 * k, 128)], sem)
            for k in range(NSETS)
        ]
        cps.append(pltpu.async_copy(emb.at[iidx_v.at[pl.ds(ch * 2 * CB, CB)]],
                                    gv, sem))
        for c in cps:
            c.wait()
        for bl in range(CB):
            eu = [uv[pl.ds(bl * D + 16 * k, 16)] for k in range(4)]
            ev = [iv[pl.ds(bl * D + 16 * k, 16)] + gv[bl, pl.ds(16 * k, 16)]
                  for k in range(4)]
            for s in range(NSETS):
                zo = bl * 128 + s * T
                rb = s * CB * T + bl * T
                v0 = zv[pl.ds(zo, 16)]
                v1 = zv[pl.ds(zo + 16, 16)]
                e0 = jnp.exp(1.0 / (1.0 + jnp.exp(-v0)))
                e1 = jnp.exp(1.0 / (1.0 + jnp.exp(-v1)))
                tot = jnp.sum(e0 + e1)
                w0 = e0 / tot
                w1 = e1 / tot
                acc = [jnp.zeros((16,), jnp.float32) for _ in range(4)]
                for t in range(T):
                    wt = w0[t] if t < 16 else w1[t - 16]
                    for k in range(4):
                        acc[k] = acc[k] + wt * rows_v[rb + t, pl.ds(16 * k, 16)]
                if s < 2:
                    eu = [eu[k] + acc[k] for k in range(4)]
                else:
                    ev = [ev[k] + acc[k] for k in range(4)]
            dotv = eu[0] * ev[0]
            for k in range(1, 4):
                dotv = dotv + eu[k] * ev[k]
            dot = jnp.sum(dotv)
            scv = 1.0 / (1.0 + jnp.exp(-jnp.full((16,), dot, jnp.float32)))
            lane = (ch % 4) * CB + bl
            svec = jnp.where(lax.iota(jnp.int32, 16) == lane, scv, svec)
        svec_ref[...] = svec
        pltpu.sync_copy(svec_ref,
                        out.at[pl.ds(w * bpw + (ch // 4) * 16, 16)])
        return svec

    lax.fori_loop(0, NCH, chunk, jnp.zeros((16,), jnp.float32))


def kernel(items, user_h_0, user_r_0, user_t_0, user_h_1, user_r_1, user_t_1,
           item_h_0, item_r_0, item_t_0, item_h_1, item_r_1, item_t_1,
           entity_emb, relation_emb, W1, W2, W3):
    r1t = _r1_call(relation_emb, W1)
    pk32, pk16 = _repack_call(entity_emb.T)
    emb_lin = pk32.reshape(NPKB * EB, D)
    emb_bf = pk16.reshape(NPKB * EB, D)

    h_idx = _pidx(jnp.concatenate([
        user_h_0.T.reshape(-1), user_h_1.T.reshape(-1),
        item_h_0.T.reshape(-1), item_h_1.T.reshape(-1),
    ]).astype(jnp.int32))
    r_idx = jnp.concatenate([
        user_r_0.T.reshape(-1), user_r_1.T.reshape(-1),
        item_r_0.T.reshape(-1), item_r_1.T.reshape(-1),
    ]).astype(jnp.int32)
    gr = _gather_rows(r1t, r_idx)
    gh = _gather_rows(emb_bf, h_idx)

    zero = jnp.zeros((D, D), jnp.bfloat16)
    w1a = W1[:D, :].astype(jnp.bfloat16)
    w1d = jnp.block([[w1a, zero], [zero, w1a]])
    w2b = W2.astype(jnp.bfloat16)
    w2d = jnp.block([[w2b, zero], [zero, w2b]])
    zc = jnp.zeros((D, 1), jnp.bfloat16)
    w3b = W3.astype(jnp.bfloat16)
    w3d = jnp.block([[w3b, zc], [zc, w3b]])
    z, uhm, ihm = _att_call(gh.reshape(N // 2, 2 * D),
                            gr.reshape(N // 2, 2 * D), w1d, w2d, w3d)

    tix = _pidx(jnp.stack([user_t_0, user_t_1, item_t_0, item_t_1]
                          ).astype(jnp.int32).reshape(NSETS * B * T))
    items2 = jnp.pad(_pidx(items.astype(jnp.int32)).reshape(B // CB, CB),
                     ((0, 0), (0, CB))).reshape(2 * B)
    return _finisher(emb_lin, tix, z.reshape(B * 2 * D),
                     uhm.reshape(B * D), ihm.reshape(B * D), items2)


# speedup vs baseline: 1.5761x; 1.5761x over previous
import functools

import jax
import jax.numpy as jnp
from jax import lax
from jax.experimental import pallas as pl
from jax.experimental.pallas import tpu as pltpu
from jax.experimental.pallas import tpu_sc as plsc

NC, NS = 2, 16
NW = NC * NS
B, T, D, NREL = 4096, 32, 64, 200
NSETS = 4
N = NSETS * T * B
RPW = N // NW
GCH = 4
GIT = RPW // (GCH * 128)

BB = 2048
NB = B // BB

CB = 4
NCH = (B // NW) // CB

_SC_PARAMS = dict(
    mesh=plsc.VectorSubcoreMesh(core_axis_name="c", subcore_axis_name="s"),
    compiler_params=pltpu.CompilerParams(
        use_tc_tiling_on_sc=False, needs_layout_passes=False),
)


def _wid():
    return lax.axis_index("s") * NC + lax.axis_index("c")


NENT = 1000000
EB = 8192
NPKB = (NENT + EB - 1) // EB


def _repack_body(et_ref, out_ref):
    out_ref[:, 0:D] = jnp.transpose(et_ref[:, 0:EB // 2])
    out_ref[:, D:2 * D] = jnp.transpose(et_ref[:, EB // 2:EB])


def _repack_call(et):
    return pl.pallas_call(
        _repack_body,
        grid=(NPKB,),
        in_specs=[pl.BlockSpec((D, EB), lambda i: (0, i))],
        out_specs=pl.BlockSpec((EB // 2, 2 * D), lambda i: (i, 0)),
        out_shape=jax.ShapeDtypeStruct((NPKB * (EB // 2), 2 * D), jnp.float32),
    )(et)


def _pidx(e):
    g = e // EB
    l = e % EB
    return 2 * (g * (EB // 2) + l % (EB // 2)) + l // (EB // 2)


def _r1_body(rel_ref, w1_ref, out_ref):
    out_ref[...] = jnp.dot(rel_ref[...], w1_ref[D:2 * D, :],
                           preferred_element_type=jnp.float32,
                           precision=lax.Precision.HIGHEST)

def _r1_call(rel, w1):
    return pl.pallas_call(
        _r1_body,
        out_shape=jax.ShapeDtypeStruct((NREL, D), jnp.float32),
    )(rel, w1)


CHN = GCH * 128


@functools.partial(
    pl.kernel,
    out_type=jax.ShapeDtypeStruct((N, D), jnp.float32),
    scratch_types=[
        pltpu.VMEM((CHN,), jnp.int32),
        pltpu.VMEM((CHN,), jnp.int32),
        pltpu.VMEM((CHN, D), jnp.float32),
        pltpu.VMEM((CHN, D), jnp.float32),
        pltpu.SemaphoreType.DMA,
        pltpu.SemaphoreType.DMA,
        pltpu.SemaphoreType.DMA,
    ],
    **_SC_PARAMS,
)
def _gather_rows(tbl, idx, out, ix0, ix1, rv0, rv1, semg, semw0, semw1):
    w = _wid()
    base = w * RPW
    last = base + (GIT - 1) * CHN

    def fire(ixb, rvb):
        return [pltpu.async_copy(tbl.at[ixb.at[pl.ds(128 * k, 128)]],
                                 rvb.at[pl.ds(128 * k, 128)], semg)
                for k in range(GCH)]

    pltpu.sync_copy(idx.at[pl.ds(base, CHN)], ix0)
    c0 = fire(ix0, rv0)
    pltpu.sync_copy(idx.at[pl.ds(base + CHN, CHN)], ix1)
    for c in c0:
        c.wait()
    pltpu.async_copy(rv0, out.at[pl.ds(base, CHN)], semw0)
    c1 = fire(ix1, rv1)
    pltpu.sync_copy(idx.at[pl.ds(base + 2 * CHN, CHN)], ix0)
    for c in c1:
        c.wait()
    pltpu.async_copy(rv1, out.at[pl.ds(base + CHN, CHN)], semw1)

    def step(m, carry):
        g0 = base + 2 * m * CHN
        g1 = g0 + CHN
        pltpu.make_async_copy(rv0, out.at[pl.ds(g0, CHN)], semw0).wait()
        ca = fire(ix0, rv0)
        pltpu.sync_copy(idx.at[pl.ds(g1, CHN)], ix1)
        for c in ca:
            c.wait()
        pltpu.async_copy(rv0, out.at[pl.ds(g0, CHN)], semw0)
        pltpu.make_async_copy(rv1, out.at[pl.ds(g1, CHN)], semw1).wait()
        cb = fire(ix1, rv1)
        nxt2 = jnp.minimum(g1 + CHN, last)
        pltpu.sync_copy(idx.at[pl.ds(nxt2, CHN)], ix0)
        for c in cb:
            c.wait()
        pltpu.async_copy(rv1, out.at[pl.ds(g1, CHN)], semw1)
        return carry

    lax.fori_loop(1, GIT // 2, step, 0)
    pltpu.make_async_copy(rv0, out.at[pl.ds(last - CHN, CHN)], semw0).wait()
    pltpu.make_async_copy(rv1, out.at[pl.ds(last, CHN)], semw1).wait()


def _att_body(gh_ref, gr_ref, w1d_ref, w2d_ref, w3d_ref,
              z_ref, uhm_ref, ihm_ref):
    j = pl.program_id(1)

    h = gh_ref[...]
    x = jnp.dot(h.astype(jnp.bfloat16), w1d_ref[...],
                preferred_element_type=jnp.float32)
    x = jnp.maximum(x + gr_ref[...], 0.0)
    x = jnp.maximum(jnp.dot(x.astype(jnp.bfloat16), w2d_ref[...],
                            preferred_element_type=jnp.float32), 0.0)
    zp = jnp.dot(x.astype(jnp.bfloat16), w3d_ref[...],
                 preferred_element_type=jnp.float32)
    iot = lax.broadcasted_iota(jnp.int32, (1, 4 * D), 1)
    acc = (zp[:, 0:1] * (iot == j).astype(jnp.float32)
           + zp[:, 1:2] * (iot == j + 2 * D).astype(jnp.float32))

    @pl.when(j == 0)
    def _():
        z_ref[...] = acc

    @pl.when(j > 0)
    def _():
        z_ref[...] = z_ref[...] + acc

    inv = 1.0 / T

    @pl.when(j == 0)
    def _():
        uhm_ref[...] = h * inv

    @pl.when((j >= 1) & (j < T))
    def _():
        uhm_ref[...] = uhm_ref[...] + h * inv

    @pl.when(j == 2 * T)
    def _():
        ihm_ref[...] = h * inv

    @pl.when((j > 2 * T) & (j < 3 * T))
    def _():
        ihm_ref[...] = ihm_ref[...] + h * inv


def _att_call(ghp, grp, w1d, w2d, w3d):
    bbp = BB // 2
    return pl.pallas_call(
        _att_body,
        grid=(NB, NSETS * T),
        in_specs=[
            pl.BlockSpec((bbp, 2 * D), lambda i, j: (j * NB + i, 0)),
            pl.BlockSpec((bbp, 2 * D), lambda i, j: (j * NB + i, 0)),
            pl.BlockSpec((2 * D, 2 * D), lambda i, j: (0, 0)),
            pl.BlockSpec((2 * D, 2 * D), lambda i, j: (0, 0)),
            pl.BlockSpec((2 * D, 2), lambda i, j: (0, 0)),
        ],
        out_specs=[
            pl.BlockSpec((bbp, 4 * D), lambda i, j: (i, 0)),
            pl.BlockSpec((bbp, 2 * D), lambda i, j: (i, 0)),
            pl.BlockSpec((bbp, 2 * D), lambda i, j: (i, 0)),
        ],
        out_shape=[
            jax.ShapeDtypeStruct((B // 2, 4 * D), jnp.float32),
            jax.ShapeDtypeStruct((B // 2, 2 * D), jnp.float32),
            jax.ShapeDtypeStruct((B // 2, 2 * D), jnp.float32),
        ],
    )(ghp, grp, w1d, w2d, w3d)


@functools.partial(
    pl.kernel,
    out_type=jax.ShapeDtypeStruct((B,), jnp.float32),
    scratch_types=[
        pltpu.VMEM((NSETS * 128,), jnp.int32),
        pltpu.VMEM((2 * (B // NW),), jnp.int32),
        pltpu.VMEM((NSETS * CB * T,), jnp.float32),
        pltpu.VMEM((NSETS * CB * T, D), jnp.float32),
        pltpu.VMEM((CB, D), jnp.float32),
        pltpu.VMEM((CB * D,), jnp.float32),
        pltpu.VMEM((CB * D,), jnp.float32),
        pltpu.VMEM((16,), jnp.float32),
        pltpu.SemaphoreType.DMA,
    ],
    **_SC_PARAMS,
)
def _finisher(emb, tix, zb, uhm, ihm, items2, out,
              idx_v, iidx_v, zv, rows_v, gv, uv, iv, svec_ref, sem):
    w = _wid()
    bpw = B // NW
    pltpu.sync_copy(items2.at[pl.ds(w * 2 * bpw, 2 * bpw)], iidx_v)

    def chunk(ch, svec):
        b0 = w * bpw + ch * CB
        for s in range(NSETS):
            pltpu.sync_copy(tix.at[pl.ds(s * B * T + b0 * T, CB * T)],
                            idx_v.at[pl.ds(s * 128, 128)])
        pltpu.sync_copy(zb.at[pl.ds(b0 * 128, CB * 128)], zv)
        pltpu.sync_copy(uhm.at[pl.ds(b0 * D, CB * D)], uv)
        pltpu.sync_copy(ihm.at[pl.ds(b0 * D, CB * D)], iv)
        cps = [
            pltpu.async_copy(emb.at[idx_v.at[pl.ds(128 * k, 128)]],
                             rows_v.at[pl.ds(128 * k, 128)], sem)
            for k in range(NSETS)
        ]
        cps.append(pltpu.async_copy(emb.at[iidx_v.at[pl.ds(ch * 2 * CB, CB)]],
                                    gv, sem))
        for c in cps:
            c.wait()
        for bl in range(CB):
            eu = [uv[pl.ds(bl * D + 16 * k, 16)] for k in range(4)]
            ev = [iv[pl.ds(bl * D + 16 * k, 16)] + gv[bl, pl.ds(16 * k, 16)]
                  for k in range(4)]
            for s in range(NSETS):
                zo = bl * 128 + s * T
                rb = s * CB * T + bl * T
                v0 = zv[pl.ds(zo, 16)]
                v1 = zv[pl.ds(zo + 16, 16)]
                e0 = jnp.exp(1.0 / (1.0 + jnp.exp(-v0)))
                e1 = jnp.exp(1.0 / (1.0 + jnp.exp(-v1)))
                tot = jnp.sum(e0 + e1)
                w0 = e0 / tot
                w1 = e1 / tot
                acc = [jnp.zeros((16,), jnp.float32) for _ in range(4)]
                for t in range(T):
                    wt = w0[t] if t < 16 else w1[t - 16]
                    for k in range(4):
                        acc[k] = acc[k] + wt * rows_v[rb + t, pl.ds(16 * k, 16)]
                if s < 2:
                    eu = [eu[k] + acc[k] for k in range(4)]
                else:
                    ev = [ev[k] + acc[k] for k in range(4)]
            dotv = eu[0] * ev[0]
            for k in range(1, 4):
                dotv = dotv + eu[k] * ev[k]
            dot = jnp.sum(dotv)
            scv = 1.0 / (1.0 + jnp.exp(-jnp.full((16,), dot, jnp.float32)))
            lane = (ch % 4) * CB + bl
            svec = jnp.where(lax.iota(jnp.int32, 16) == lane, scv, svec)
        svec_ref[...] = svec
        pltpu.sync_copy(svec_ref,
                        out.at[pl.ds(w * bpw + (ch // 4) * 16, 16)])
        return svec

    lax.fori_loop(0, NCH, chunk, jnp.zeros((16,), jnp.float32))


def kernel(items, user_h_0, user_r_0, user_t_0, user_h_1, user_r_1, user_t_1,
           item_h_0, item_r_0, item_t_0, item_h_1, item_r_1, item_t_1,
           entity_emb, relation_emb, W1, W2, W3):
    r1t = _r1_call(relation_emb, W1)
    emb_lin = _repack_call(entity_emb.T).reshape(NPKB * EB, D)

    h_idx = _pidx(jnp.concatenate([
        user_h_0.T.reshape(-1), user_h_1.T.reshape(-1),
        item_h_0.T.reshape(-1), item_h_1.T.reshape(-1),
    ]).astype(jnp.int32))
    r_idx = jnp.concatenate([
        user_r_0.T.reshape(-1), user_r_1.T.reshape(-1),
        item_r_0.T.reshape(-1), item_r_1.T.reshape(-1),
    ]).astype(jnp.int32)
    gr = _gather_rows(r1t, r_idx)
    gh = _gather_rows(emb_lin, h_idx)

    zero = jnp.zeros((D, D), jnp.bfloat16)
    w1a = W1[:D, :].astype(jnp.bfloat16)
    w1d = jnp.block([[w1a, zero], [zero, w1a]])
    w2b = W2.astype(jnp.bfloat16)
    w2d = jnp.block([[w2b, zero], [zero, w2b]])
    zc = jnp.zeros((D, 1), jnp.bfloat16)
    w3b = W3.astype(jnp.bfloat16)
    w3d = jnp.block([[w3b, zc], [zc, w3b]])
    z, uhm, ihm = _att_call(gh.reshape(N // 2, 2 * D),
                            gr.reshape(N // 2, 2 * D), w1d, w2d, w3d)

    tix = _pidx(jnp.stack([user_t_0, user_t_1, item_t_0, item_t_1]
                          ).astype(jnp.int32).reshape(NSETS * B * T))
    items2 = jnp.pad(_pidx(items.astype(jnp.int32)).reshape(B // CB, CB),
                     ((0, 0), (0, CB))).reshape(2 * B)
    return _finisher(emb_lin, tix, z.reshape(B * 2 * D),
                     uhm.reshape(B * D), ihm.reshape(B * D), items2)


# speedup vs baseline: 1.6826x; 1.0676x over previous
import functools

import jax
import jax.numpy as jnp
from jax import lax
from jax.experimental import pallas as pl
from jax.experimental.pallas import tpu as pltpu
from jax.experimental.pallas import tpu_sc as plsc

NC, NS = 2, 16
NW = NC * NS
B, T, D, NREL = 4096, 32, 64, 200
NSETS = 4
N = NSETS * T * B
RPW = N // NW
GCH = 4
GIT = RPW // (GCH * 128)

BB = 4096
NB = B // BB

CB = 4
NCH = (B // NW) // CB

_SC_PARAMS = dict(
    mesh=plsc.VectorSubcoreMesh(core_axis_name="c", subcore_axis_name="s"),
    compiler_params=pltpu.CompilerParams(
        use_tc_tiling_on_sc=False, needs_layout_passes=False),
)


def _wid():
    return lax.axis_index("s") * NC + lax.axis_index("c")


NENT = 1000000
EB = 8192
NPKB = (NENT + EB - 1) // EB


def _repack_body(et_ref, out_ref):
    out_ref[:, 0:D] = jnp.transpose(et_ref[:, 0:EB // 2])
    out_ref[:, D:2 * D] = jnp.transpose(et_ref[:, EB // 2:EB])


def _repack_call(et):
    return pl.pallas_call(
        _repack_body,
        grid=(NPKB,),
        in_specs=[pl.BlockSpec((D, EB), lambda i: (0, i))],
        out_specs=pl.BlockSpec((EB // 2, 2 * D), lambda i: (i, 0)),
        out_shape=jax.ShapeDtypeStruct((NPKB * (EB // 2), 2 * D), jnp.float32),
    )(et)


def _pidx(e):
    g = e // EB
    l = e % EB
    return 2 * (g * (EB // 2) + l % (EB // 2)) + l // (EB // 2)


def _r1_body(rel_ref, w1_ref, out_ref):
    out_ref[...] = jnp.dot(rel_ref[...], w1_ref[D:2 * D, :],
                           preferred_element_type=jnp.float32,
                           precision=lax.Precision.HIGHEST)

def _r1_call(rel, w1):
    return pl.pallas_call(
        _r1_body,
        out_shape=jax.ShapeDtypeStruct((NREL, D), jnp.float32),
    )(rel, w1)


CHN = GCH * 128


@functools.partial(
    pl.kernel,
    out_type=jax.ShapeDtypeStruct((N, D), jnp.float32),
    scratch_types=[
        pltpu.VMEM((CHN,), jnp.int32),
        pltpu.VMEM((CHN,), jnp.int32),
        pltpu.VMEM((CHN, D), jnp.float32),
        pltpu.VMEM((CHN, D), jnp.float32),
        pltpu.SemaphoreType.DMA,
        pltpu.SemaphoreType.DMA,
        pltpu.SemaphoreType.DMA,
    ],
    **_SC_PARAMS,
)
def _gather_rows(tbl, idx, out, ix0, ix1, rv0, rv1, semg, semw0, semw1):
    w = _wid()
    base = w * RPW
    last = base + (GIT - 1) * CHN

    def fire(ixb, rvb):
        return [pltpu.async_copy(tbl.at[ixb.at[pl.ds(128 * k, 128)]],
                                 rvb.at[pl.ds(128 * k, 128)], semg)
                for k in range(GCH)]

    pltpu.sync_copy(idx.at[pl.ds(base, CHN)], ix0)
    c0 = fire(ix0, rv0)
    pltpu.sync_copy(idx.at[pl.ds(base + CHN, CHN)], ix1)
    for c in c0:
        c.wait()
    pltpu.async_copy(rv0, out.at[pl.ds(base, CHN)], semw0)
    c1 = fire(ix1, rv1)
    pltpu.sync_copy(idx.at[pl.ds(base + 2 * CHN, CHN)], ix0)
    for c in c1:
        c.wait()
    pltpu.async_copy(rv1, out.at[pl.ds(base + CHN, CHN)], semw1)

    def step(m, carry):
        g0 = base + 2 * m * CHN
        g1 = g0 + CHN
        pltpu.make_async_copy(rv0, out.at[pl.ds(g0, CHN)], semw0).wait()
        ca = fire(ix0, rv0)
        pltpu.sync_copy(idx.at[pl.ds(g1, CHN)], ix1)
        for c in ca:
            c.wait()
        pltpu.async_copy(rv0, out.at[pl.ds(g0, CHN)], semw0)
        pltpu.make_async_copy(rv1, out.at[pl.ds(g1, CHN)], semw1).wait()
        cb = fire(ix1, rv1)
        nxt2 = jnp.minimum(g1 + CHN, last)
        pltpu.sync_copy(idx.at[pl.ds(nxt2, CHN)], ix0)
        for c in cb:
            c.wait()
        pltpu.async_copy(rv1, out.at[pl.ds(g1, CHN)], semw1)
        return carry

    lax.fori_loop(1, GIT // 2, step, 0)
    pltpu.make_async_copy(rv0, out.at[pl.ds(last - CHN, CHN)], semw0).wait()
    pltpu.make_async_copy(rv1, out.at[pl.ds(last, CHN)], semw1).wait()


def _att_body(gh_ref, gr_ref, w1d_ref, w2d_ref, w3d_ref,
              z_ref, uhm_ref, ihm_ref):
    j = pl.program_id(1)

    h = gh_ref[...]
    x = jnp.dot(h.astype(jnp.bfloat16), w1d_ref[...],
                preferred_element_type=jnp.float32)
    x = jnp.maximum(x + gr_ref[...], 0.0)
    x = jnp.maximum(jnp.dot(x.astype(jnp.bfloat16), w2d_ref[...],
                            preferred_element_type=jnp.float32), 0.0)
    zp = jnp.dot(x.astype(jnp.bfloat16), w3d_ref[...],
                 preferred_element_type=jnp.float32)
    iot = lax.broadcasted_iota(jnp.int32, (1, 4 * D), 1)
    acc = (zp[:, 0:1] * (iot == j).astype(jnp.float32)
           + zp[:, 1:2] * (iot == j + 2 * D).astype(jnp.float32))

    @pl.when(j == 0)
    def _():
        z_ref[...] = acc

    @pl.when(j > 0)
    def _():
        z_ref[...] = z_ref[...] + acc

    inv = 1.0 / T

    @pl.when(j == 0)
    def _():
        uhm_ref[...] = h * inv

    @pl.when((j >= 1) & (j < T))
    def _():
        uhm_ref[...] = uhm_ref[...] + h * inv

    @pl.when(j == 2 * T)
    def _():
        ihm_ref[...] = h * inv

    @pl.when((j > 2 * T) & (j < 3 * T))
    def _():
        ihm_ref[...] = ihm_ref[...] + h * inv


def _att_call(ghp, grp, w1d, w2d, w3d):
    bbp = BB // 2
    return pl.pallas_call(
        _att_body,
        grid=(NB, NSETS * T),
        in_specs=[
            pl.BlockSpec((bbp, 2 * D), lambda i, j: (j * NB + i, 0)),
            pl.BlockSpec((bbp, 2 * D), lambda i, j: (j * NB + i, 0)),
            pl.BlockSpec((2 * D, 2 * D), lambda i, j: (0, 0)),
            pl.BlockSpec((2 * D, 2 * D), lambda i, j: (0, 0)),
            pl.BlockSpec((2 * D, 2), lambda i, j: (0, 0)),
        ],
        out_specs=[
            pl.BlockSpec((bbp, 4 * D), lambda i, j: (i, 0)),
            pl.BlockSpec((bbp, 2 * D), lambda i, j: (i, 0)),
            pl.BlockSpec((bbp, 2 * D), lambda i, j: (i, 0)),
        ],
        out_shape=[
            jax.ShapeDtypeStruct((B // 2, 4 * D), jnp.float32),
            jax.ShapeDtypeStruct((B // 2, 2 * D), jnp.float32),
            jax.ShapeDtypeStruct((B // 2, 2 * D), jnp.float32),
        ],
    )(ghp, grp, w1d, w2d, w3d)


@functools.partial(
    pl.kernel,
    out_type=jax.ShapeDtypeStruct((B,), jnp.float32),
    scratch_types=[
        pltpu.VMEM((NSETS * 128,), jnp.int32),
        pltpu.VMEM((2 * (B // NW),), jnp.int32),
        pltpu.VMEM((NSETS * CB * T,), jnp.float32),
        pltpu.VMEM((NSETS * CB * T, D), jnp.float32),
        pltpu.VMEM((CB, D), jnp.float32),
        pltpu.VMEM((CB * D,), jnp.float32),
        pltpu.VMEM((CB * D,), jnp.float32),
        pltpu.VMEM((16,), jnp.float32),
        pltpu.SemaphoreType.DMA,
    ],
    **_SC_PARAMS,
)
def _finisher(emb, tix, zb, uhm, ihm, items2, out,
              idx_v, iidx_v, zv, rows_v, gv, uv, iv, svec_ref, sem):
    w = _wid()
    bpw = B // NW
    pltpu.sync_copy(items2.at[pl.ds(w * 2 * bpw, 2 * bpw)], iidx_v)

    def chunk(ch, svec):
        b0 = w * bpw + ch * CB
        for s in range(NSETS):
            pltpu.sync_copy(tix.at[pl.ds(s * B * T + b0 * T, CB * T)],
                            idx_v.at[pl.ds(s * 128, 128)])
        pltpu.sync_copy(zb.at[pl.ds(b0 * 128, CB * 128)], zv)
        pltpu.sync_copy(uhm.at[pl.ds(b0 * D, CB * D)], uv)
        pltpu.sync_copy(ihm.at[pl.ds(b0 * D, CB * D)], iv)
        cps = [
            pltpu.async_copy(emb.at[idx_v.at[pl.ds(128 * k, 128)]],
                             rows_v.at[pl.ds(128 * k, 128)], sem)
            for k in range(NSETS)
        ]
        cps.append(pltpu.async_copy(emb.at[iidx_v.at[pl.ds(ch * 2 * CB, CB)]],
                                    gv, sem))
        for c in cps:
            c.wait()
        for bl in range(CB):
            eu = [uv[pl.ds(bl * D + 16 * k, 16)] for k in range(4)]
            ev = [iv[pl.ds(bl * D + 16 * k, 16)] + gv[bl, pl.ds(16 * k, 16)]
                  for k in range(4)]
            for s in range(NSETS):
                zo = bl * 128 + s * T
                rb = s * CB * T + bl * T
                v0 = zv[pl.ds(zo, 16)]
                v1 = zv[pl.ds(zo + 16, 16)]
                e0 = jnp.exp(1.0 / (1.0 + jnp.exp(-v0)))
                e1 = jnp.exp(1.0 / (1.0 + jnp.exp(-v1)))
                tot = jnp.sum(e0 + e1)
                w0 = e0 / tot
                w1 = e1 / tot
                acc = [jnp.zeros((16,), jnp.float32) for _ in range(4)]
                for t in range(T):
                    wt = w0[t] if t < 16 else w1[t - 16]
                    for k in range(4):
                        acc[k] = acc[k] + wt * rows_v[rb + t, pl.ds(16 * k, 16)]
                if s < 2:
                    eu = [eu[k] + acc[k] for k in range(4)]
                else:
                    ev = [ev[k] + acc[k] for k in range(4)]
            dotv = eu[0] * ev[0]
            for k in range(1, 4):
                dotv = dotv + eu[k] * ev[k]
            dot = jnp.sum(dotv)
            scv = 1.0 / (1.0 + jnp.exp(-jnp.full((16,), dot, jnp.float32)))
            lane = (ch % 4) * CB + bl
            svec = jnp.where(lax.iota(jnp.int32, 16) == lane, scv, svec)
        svec_ref[...] = svec
        pltpu.sync_copy(svec_ref,
                        out.at[pl.ds(w * bpw + (ch // 4) * 16, 16)])
        return svec

    lax.fori_loop(0, NCH, chunk, jnp.zeros((16,), jnp.float32))


def kernel(items, user_h_0, user_r_0, user_t_0, user_h_1, user_r_1, user_t_1,
           item_h_0, item_r_0, item_t_0, item_h_1, item_r_1, item_t_1,
           entity_emb, relation_emb, W1, W2, W3):
    r1t = _r1_call(relation_emb, W1)
    emb_lin = _repack_call(entity_emb.T).reshape(NPKB * EB, D)

    h_idx = _pidx(jnp.concatenate([
        user_h_0.T.reshape(-1), user_h_1.T.reshape(-1),
        item_h_0.T.reshape(-1), item_h_1.T.reshape(-1),
    ]).astype(jnp.int32))
    r_idx = jnp.concatenate([
        user_r_0.T.reshape(-1), user_r_1.T.reshape(-1),
        item_r_0.T.reshape(-1), item_r_1.T.reshape(-1),
    ]).astype(jnp.int32)
    gr = _gather_rows(r1t, r_idx)
    gh = _gather_rows(emb_lin, h_idx)

    zero = jnp.zeros((D, D), jnp.bfloat16)
    w1a = W1[:D, :].astype(jnp.bfloat16)
    w1d = jnp.block([[w1a, zero], [zero, w1a]])
    w2b = W2.astype(jnp.bfloat16)
    w2d = jnp.block([[w2b, zero], [zero, w2b]])
    zc = jnp.zeros((D, 1), jnp.bfloat16)
    w3b = W3.astype(jnp.bfloat16)
    w3d = jnp.block([[w3b, zc], [zc, w3b]])
    z, uhm, ihm = _att_call(gh.reshape(N // 2, 2 * D),
                            gr.reshape(N // 2, 2 * D), w1d, w2d, w3d)

    tix = _pidx(jnp.stack([user_t_0, user_t_1, item_t_0, item_t_1]
                          ).astype(jnp.int32).reshape(NSETS * B * T))
    items2 = jnp.pad(_pidx(items.astype(jnp.int32)).reshape(B // CB, CB),
                     ((0, 0), (0, CB))).reshape(2 * B)
    return _finisher(emb_lin, tix, z.reshape(B * 2 * D),
                     uhm.reshape(B * D), ihm.reshape(B * D), items2)


# speedup vs baseline: 1.7959x; 1.0673x over previous
import functools

import jax
import jax.numpy as jnp
from jax import lax
from jax.experimental import pallas as pl
from jax.experimental.pallas import tpu as pltpu
from jax.experimental.pallas import tpu_sc as plsc

NC, NS = 2, 16
NW = NC * NS
B, T, D, NREL = 4096, 32, 64, 200
NSETS = 4
N = NSETS * T * B
RPW = N // NW
GCH = 4
GIT = RPW // (GCH * 128)

BB = 4096
NB = B // BB

CB = 4
NCH = (B // NW) // CB

_SC_PARAMS = dict(
    mesh=plsc.VectorSubcoreMesh(core_axis_name="c", subcore_axis_name="s"),
    compiler_params=pltpu.CompilerParams(
        use_tc_tiling_on_sc=False, needs_layout_passes=False),
)


def _wid():
    return lax.axis_index("s") * NC + lax.axis_index("c")


NENT = 1000000
EB = 8192
NPKB = (NENT + EB - 1) // EB


def _repack_body(et_ref, out_ref):
    out_ref[:, 0:D] = jnp.transpose(et_ref[:, 0:EB // 2])
    out_ref[:, D:2 * D] = jnp.transpose(et_ref[:, EB // 2:EB])


def _repack_call(et):
    return pl.pallas_call(
        _repack_body,
        grid=(NPKB,),
        in_specs=[pl.BlockSpec((D, EB), lambda i: (0, i))],
        out_specs=pl.BlockSpec((EB // 2, 2 * D), lambda i: (i, 0)),
        out_shape=jax.ShapeDtypeStruct((NPKB * (EB // 2), 2 * D), jnp.float32),
    )(et)


def _pidx(e):
    g = e // EB
    l = e % EB
    return 2 * (g * (EB // 2) + l % (EB // 2)) + l // (EB // 2)


def _r1_body(rel_ref, w1_ref, out_ref):
    out_ref[...] = jnp.dot(rel_ref[...], w1_ref[D:2 * D, :],
                           preferred_element_type=jnp.float32,
                           precision=lax.Precision.HIGHEST)

def _r1_call(rel, w1):
    return pl.pallas_call(
        _r1_body,
        out_shape=jax.ShapeDtypeStruct((NREL, D), jnp.float32),
    )(rel, w1)


CHN = GCH * 128


@functools.partial(
    pl.kernel,
    out_type=jax.ShapeDtypeStruct((N, D), jnp.float32),
    scratch_types=[
        pltpu.VMEM((CHN,), jnp.int32),
        pltpu.VMEM((CHN,), jnp.int32),
        pltpu.VMEM((CHN, D), jnp.float32),
        pltpu.VMEM((CHN, D), jnp.float32),
        pltpu.SemaphoreType.DMA,
        pltpu.SemaphoreType.DMA,
        pltpu.SemaphoreType.DMA,
    ],
    **_SC_PARAMS,
)
def _gather_rows(tbl, idx, out, ix0, ix1, rv0, rv1, semg, semw0, semw1):
    w = _wid()
    base = w * RPW
    last = base + (GIT - 1) * CHN

    def fire(ixb, rvb):
        return [pltpu.async_copy(tbl.at[ixb.at[pl.ds(128 * k, 128)]],
                                 rvb.at[pl.ds(128 * k, 128)], semg)
                for k in range(GCH)]

    pltpu.sync_copy(idx.at[pl.ds(base, CHN)], ix0)
    c0 = fire(ix0, rv0)
    pltpu.sync_copy(idx.at[pl.ds(base + CHN, CHN)], ix1)
    for c in c0:
        c.wait()
    pltpu.async_copy(rv0, out.at[pl.ds(base, CHN)], semw0)
    c1 = fire(ix1, rv1)
    pltpu.sync_copy(idx.at[pl.ds(base + 2 * CHN, CHN)], ix0)
    for c in c1:
        c.wait()
    pltpu.async_copy(rv1, out.at[pl.ds(base + CHN, CHN)], semw1)

    def step(m, carry):
        g0 = base + 2 * m * CHN
        g1 = g0 + CHN
        pltpu.make_async_copy(rv0, out.at[pl.ds(g0, CHN)], semw0).wait()
        ca = fire(ix0, rv0)
        pltpu.sync_copy(idx.at[pl.ds(g1, CHN)], ix1)
        for c in ca:
            c.wait()
        pltpu.async_copy(rv0, out.at[pl.ds(g0, CHN)], semw0)
        pltpu.make_async_copy(rv1, out.at[pl.ds(g1, CHN)], semw1).wait()
        cb = fire(ix1, rv1)
        nxt2 = jnp.minimum(g1 + CHN, last)
        pltpu.sync_copy(idx.at[pl.ds(nxt2, CHN)], ix0)
        for c in cb:
            c.wait()
        pltpu.async_copy(rv1, out.at[pl.ds(g1, CHN)], semw1)
        return carry

    lax.fori_loop(1, GIT // 2, step, 0)
    pltpu.make_async_copy(rv0, out.at[pl.ds(last - CHN, CHN)], semw0).wait()
    pltpu.make_async_copy(rv1, out.at[pl.ds(last, CHN)], semw1).wait()


def _att_body(gh_ref, gr_ref, w1d_ref, w2d_ref, w3d_ref,
              z_ref, uhm_ref, ihm_ref):
    j = pl.program_id(1)

    h = gh_ref[...]
    x = jnp.dot(h.astype(jnp.bfloat16), w1d_ref[...],
                preferred_element_type=jnp.float32)
    x = jnp.maximum(x + gr_ref[...], 0.0)
    x = jnp.maximum(jnp.dot(x.astype(jnp.bfloat16), w2d_ref[...],
                            preferred_element_type=jnp.float32), 0.0)
    zp = jnp.dot(x.astype(jnp.bfloat16), w3d_ref[...],
                 preferred_element_type=jnp.float32)
    iot = lax.broadcasted_iota(jnp.int32, (1, 4 * D), 1)
    acc = (zp[:, 0:1] * (iot == j).astype(jnp.float32)
           + zp[:, 1:2] * (iot == j + 2 * D).astype(jnp.float32))

    @pl.when(j == 0)
    def _():
        z_ref[...] = acc

    @pl.when(j > 0)
    def _():
        z_ref[...] = z_ref[...] + acc

    inv = 1.0 / T

    @pl.when(j == 0)
    def _():
        uhm_ref[...] = h * inv

    @pl.when((j >= 1) & (j < T))
    def _():
        uhm_ref[...] = uhm_ref[...] + h * inv

    @pl.when(j == 2 * T)
    def _():
        ihm_ref[...] = h * inv

    @pl.when((j > 2 * T) & (j < 3 * T))
    def _():
        ihm_ref[...] = ihm_ref[...] + h * inv


def _att_call(ghp, grp, w1d, w2d, w3d):
    bbp = BB // 2
    return pl.pallas_call(
        _att_body,
        grid=(NB, NSETS * T),
        in_specs=[
            pl.BlockSpec((bbp, 2 * D), lambda i, j: (j * NB + i, 0)),
            pl.BlockSpec((bbp, 2 * D), lambda i, j: (j * NB + i, 0)),
            pl.BlockSpec((2 * D, 2 * D), lambda i, j: (0, 0)),
            pl.BlockSpec((2 * D, 2 * D), lambda i, j: (0, 0)),
            pl.BlockSpec((2 * D, 2), lambda i, j: (0, 0)),
        ],
        out_specs=[
            pl.BlockSpec((bbp, 4 * D), lambda i, j: (i, 0)),
            pl.BlockSpec((bbp, 2 * D), lambda i, j: (i, 0)),
            pl.BlockSpec((bbp, 2 * D), lambda i, j: (i, 0)),
        ],
        out_shape=[
            jax.ShapeDtypeStruct((B // 2, 4 * D), jnp.float32),
            jax.ShapeDtypeStruct((B // 2, 2 * D), jnp.float32),
            jax.ShapeDtypeStruct((B // 2, 2 * D), jnp.float32),
        ],
    )(ghp, grp, w1d, w2d, w3d)


@functools.partial(
    pl.kernel,
    out_type=jax.ShapeDtypeStruct((B,), jnp.float32),
    scratch_types=[
        pltpu.VMEM((NSETS * 128,), jnp.int32),
        pltpu.VMEM((2 * (B // NW),), jnp.int32),
        pltpu.VMEM((NSETS * CB * T,), jnp.float32),
        pltpu.VMEM((NSETS * CB * T, D), jnp.float32),
        pltpu.VMEM((CB, D), jnp.float32),
        pltpu.VMEM((CB * D,), jnp.float32),
        pltpu.VMEM((CB * D,), jnp.float32),
        pltpu.VMEM((16,), jnp.float32),
        pltpu.SemaphoreType.DMA,
        pltpu.SemaphoreType.DMA,
        pltpu.SemaphoreType.DMA,
    ],
    **_SC_PARAMS,
)
def _finisher(emb, tix, zb, uhm, ihm, items2, out,
              idx_v, iidx_v, zv, rows_v, gv, uv, iv, svec_ref,
              sem, semc, semd):
    w = _wid()
    bpw = B // NW
    pltpu.sync_copy(items2.at[pl.ds(w * 2 * bpw, 2 * bpw)], iidx_v)

    def chunk(ch, svec):
        b0 = w * bpw + ch * CB
        cidx = [
            pltpu.async_copy(tix.at[pl.ds(s * B * T + b0 * T, CB * T)],
                             idx_v.at[pl.ds(s * 128, 128)], semc)
            for s in range(NSETS)
        ]
        csml = [
            pltpu.async_copy(zb.at[pl.ds(b0 * 128, CB * 128)], zv, semd),
            pltpu.async_copy(uhm.at[pl.ds(b0 * D, CB * D)], uv, semd),
            pltpu.async_copy(ihm.at[pl.ds(b0 * D, CB * D)], iv, semd),
        ]
        for c in cidx:
            c.wait()
        cps = [
            pltpu.async_copy(emb.at[idx_v.at[pl.ds(128 * k, 128)]],
                             rows_v.at[pl.ds(128 * k, 128)], sem)
            for k in range(NSETS)
        ]
        cps.append(pltpu.async_copy(emb.at[iidx_v.at[pl.ds(ch * 2 * CB, CB)]],
                                    gv, sem))
        for c in csml:
            c.wait()
        for c in cps:
            c.wait()
        for bl in range(CB):
            eu = [uv[pl.ds(bl * D + 16 * k, 16)] for k in range(4)]
            ev = [iv[pl.ds(bl * D + 16 * k, 16)] + gv[bl, pl.ds(16 * k, 16)]
                  for k in range(4)]
            for s in range(NSETS):
                zo = bl * 128 + s * T
                rb = s * CB * T + bl * T
                v0 = zv[pl.ds(zo, 16)]
                v1 = zv[pl.ds(zo + 16, 16)]
                e0 = jnp.exp(1.0 / (1.0 + jnp.exp(-v0)))
                e1 = jnp.exp(1.0 / (1.0 + jnp.exp(-v1)))
                tot = jnp.sum(e0 + e1)
                w0 = e0 / tot
                w1 = e1 / tot
                acc = [jnp.zeros((16,), jnp.float32) for _ in range(4)]
                for t in range(T):
                    wt = w0[t] if t < 16 else w1[t - 16]
                    for k in range(4):
                        acc[k] = acc[k] + wt * rows_v[rb + t, pl.ds(16 * k, 16)]
                if s < 2:
                    eu = [eu[k] + acc[k] for k in range(4)]
                else:
                    ev = [ev[k] + acc[k] for k in range(4)]
            dotv = eu[0] * ev[0]
            for k in range(1, 4):
                dotv = dotv + eu[k] * ev[k]
            dot = jnp.sum(dotv)
            scv = 1.0 / (1.0 + jnp.exp(-jnp.full((16,), dot, jnp.float32)))
            lane = (ch % 4) * CB + bl
            svec = jnp.where(lax.iota(jnp.int32, 16) == lane, scv, svec)
        svec_ref[...] = svec
        pltpu.sync_copy(svec_ref,
                        out.at[pl.ds(w * bpw + (ch // 4) * 16, 16)])
        return svec

    lax.fori_loop(0, NCH, chunk, jnp.zeros((16,), jnp.float32))


def kernel(items, user_h_0, user_r_0, user_t_0, user_h_1, user_r_1, user_t_1,
           item_h_0, item_r_0, item_t_0, item_h_1, item_r_1, item_t_1,
           entity_emb, relation_emb, W1, W2, W3):
    r1t = _r1_call(relation_emb, W1)
    emb_lin = _repack_call(entity_emb.T).reshape(NPKB * EB, D)

    h_idx = _pidx(jnp.concatenate([
        user_h_0.T.reshape(-1), user_h_1.T.reshape(-1),
        item_h_0.T.reshape(-1), item_h_1.T.reshape(-1),
    ]).astype(jnp.int32))
    r_idx = jnp.concatenate([
        user_r_0.T.reshape(-1), user_r_1.T.reshape(-1),
        item_r_0.T.reshape(-1), item_r_1.T.reshape(-1),
    ]).astype(jnp.int32)
    gr = _gather_rows(r1t, r_idx)
    gh = _gather_rows(emb_lin, h_idx)

    zero = jnp.zeros((D, D), jnp.bfloat16)
    w1a = W1[:D, :].astype(jnp.bfloat16)
    w1d = jnp.block([[w1a, zero], [zero, w1a]])
    w2b = W2.astype(jnp.bfloat16)
    w2d = jnp.block([[w2b, zero], [zero, w2b]])
    zc = jnp.zeros((D, 1), jnp.bfloat16)
    w3b = W3.astype(jnp.bfloat16)
    w3d = jnp.block([[w3b, zc], [zc, w3b]])
    z, uhm, ihm = _att_call(gh.reshape(N // 2, 2 * D),
                            gr.reshape(N // 2, 2 * D), w1d, w2d, w3d)

    tix = _pidx(jnp.stack([user_t_0, user_t_1, item_t_0, item_t_1]
                          ).astype(jnp.int32).reshape(NSETS * B * T))
    items2 = jnp.pad(_pidx(items.astype(jnp.int32)).reshape(B // CB, CB),
                     ((0, 0), (0, CB))).reshape(2 * B)
    return _finisher(emb_lin, tix, z.reshape(B * 2 * D),
                     uhm.reshape(B * D), ihm.reshape(B * D), items2)
